# baseline (speedup 1.0000x reference)
import functools

import jax
import jax.numpy as jnp
from jax import lax
from jax.experimental import pallas as pl
from jax.experimental.pallas import tpu as pltpu
from jax.experimental.pallas import tpu_sc as plsc

EMBED_DIM = 32
NUM_CORES = 2
NUM_SUBCORES = 16
NUM_WORKERS = NUM_CORES * NUM_SUBCORES
B0 = 16384
B1 = 50
COLS = B0 // NUM_WORKERS
NBLK = COLS // 128


def _embed_kernel(idx_hbm, table_hbm, out_hbm, idx_blk, idx_list, rows_v,
                  t_v, isem, gsem, wsem):
    wid = lax.axis_index("s") * NUM_CORES + lax.axis_index("c")
    col0 = wid * COLS
    viota = lax.iota(jnp.int32, 16)

    def extract_idx(b1, buf):
        cols = jnp.full((16,), 0, jnp.int32) + b1
        for j in range(COLS // 16):
            rows = viota + (16 * j)
            vals = plsc.load_gather(idx_blk, [rows, cols])
            idx_list[buf, pl.ds(16 * j, 16)] = vals

    def issue_gather(buf):
        return pltpu.async_copy(
            table_hbm.at[idx_list.at[buf]], rows_v.at[buf], gsem.at[buf]
        )

    def wait_gather(buf):
        pltpu.make_async_copy(
            table_hbm.at[idx_list.at[buf]], rows_v.at[buf], gsem.at[buf]
        ).wait()

    row_vecs = [
        [viota + (bb * 128 + jl * 16) for jl in range(8)] for bb in range(NBLK)
    ]
    col_vecs = [jnp.full((16,), f, jnp.int32) for f in range(EMBED_DIM)]

    def transpose(buf):
        rows2d = rows_v.at[buf]
        for f_blk in range(4):
            for bb in range(NBLK):
                for f8 in range(8):
                    cols = col_vecs[f_blk * 8 + f8]
                    for jl in range(8):
                        vals = plsc.load_gather(
                            rows2d, [row_vecs[bb][jl], cols]
                        )
                        t_v[buf, f_blk, bb, f8, pl.ds(jl * 16, 16)] = vals

    def wait_wb(buf):
        pltpu.make_async_copy(
            t_v.at[buf], out_hbm.at[0, :, pl.ds(0, NBLK), :, :], wsem.at[buf]
        ).wait()

    pltpu.async_copy(
        idx_hbm.at[pl.ds(col0, COLS), :], idx_blk, isem
    ).wait()

    extract_idx(0, 0)
    issue_gather(0)

    def body(g, carry):
        for buf in range(2):
            i = g * 2 + buf
            wait_gather(buf)

            @pl.when(i + 1 < B1)
            def _():
                extract_idx(i + 1, 1 - buf)
                issue_gather(1 - buf)

            @pl.when(i >= 2)
            def _():
                wait_wb(buf)

            transpose(buf)
            pltpu.async_copy(
                t_v.at[buf],
                out_hbm.at[i, :, pl.ds(wid * NBLK, NBLK), :, :],
                wsem.at[buf],
            )
        return carry

    lax.fori_loop(0, B1 // 2, body, 0)
    for buf in range(2):
        wait_wb(buf)


def kernel(input, weight):
    idx = input.astype(jnp.int32)

    mesh = plsc.VectorSubcoreMesh(core_axis_name="c", subcore_axis_name="s")
    run = pl.kernel(
        _embed_kernel,
        mesh=mesh,
        out_type=jax.ShapeDtypeStruct((B1, 4, B0 // 128, 8, 128), jnp.float32),
        scratch_types=[
            pltpu.VMEM((COLS, B1), jnp.int32),
            pltpu.VMEM((2, COLS), jnp.int32),
            pltpu.VMEM((2, COLS, EMBED_DIM), jnp.float32),
            pltpu.VMEM((2, 4, NBLK, 8, 128), jnp.float32),
            pltpu.SemaphoreType.DMA,
            pltpu.SemaphoreType.DMA((2,)),
            pltpu.SemaphoreType.DMA((2,)),
        ],
        compiler_params=pltpu.CompilerParams(
            use_tc_tiling_on_sc=False, needs_layout_passes=False
        ),
    )
    out5d = run(idx, weight)
    return out5d.transpose(2, 4, 0, 1, 3).reshape(B0, B1, EMBED_DIM)


# speedup vs baseline: 1.8622x; 1.8622x over previous
import jax
import jax.numpy as jnp
from jax import lax
from jax.experimental import pallas as pl
from jax.experimental.pallas import tpu as pltpu
from jax.experimental.pallas import tpu_sc as plsc

EMBED_DIM = 32
NUM_CORES = 2
NUM_SUBCORES = 16
NUM_WORKERS = NUM_CORES * NUM_SUBCORES
B0 = 16384
B1 = 50
COLS = B0 // NUM_WORKERS
NBLK = COLS // 128
TPAD = 133


def _embed_kernel(idx_hbm, table_hbm, out_hbm, idx_blk, idx_list, rows_v,
                  t_v, isem, gsem, wsem):
    wid = lax.axis_index("s") * NUM_CORES + lax.axis_index("c")
    col0 = wid * COLS
    viota = lax.iota(jnp.int32, 16)
    p_lo = (viota // 8) * 32 + (viota % 8)
    p_hi = ((viota + 16) // 8) * 32 + (viota % 8)

    def extract_idx(b1, buf):
        cols = jnp.full((16,), 0, jnp.int32) + b1
        for j in range(COLS // 16):
            rows = viota + (16 * j)
            vals = plsc.load_gather(idx_blk, [rows, cols])
            idx_list[buf, pl.ds(16 * j, 16)] = vals

    def issue_gather(buf):
        return pltpu.async_copy(
            table_hbm.at[idx_list.at[buf]], rows_v.at[buf], gsem.at[buf]
        )

    def wait_gather(buf):
        pltpu.make_async_copy(
            table_hbm.at[idx_list.at[buf]], rows_v.at[buf], gsem.at[buf]
        ).wait()

    def transpose(buf):
        t2d = t_v.at[buf]

        def bb_body(bb, carry):
            row_lo = p_lo + bb * 8
            row_hi = p_hi + bb * 8

            def cc_body(cc, carry2):
                for k in range(16):
                    r = bb * 128 + cc * 16 + k
                    c = jnp.full((16,), 0, jnp.int32) + (cc * 16 + k)
                    v_lo = rows_v[buf, r, pl.ds(0, 16)]
                    v_hi = rows_v[buf, r, pl.ds(16, 16)]
                    plsc.store_scatter(t2d, [row_lo, c], v_lo)
                    plsc.store_scatter(t2d, [row_hi, c], v_hi)
                return carry2

            return lax.fori_loop(0, 8, cc_body, carry)

        lax.fori_loop(0, NBLK, bb_body, 0)

    def issue_wb(i, buf):
        for f_blk in range(4):
            pltpu.async_copy(
                t_v.at[buf, pl.ds(f_blk * 32, 32), pl.ds(0, 128)],
                out_hbm.at[i, pl.ds(f_blk * 1024 + wid * 32, 32), :],
                wsem.at[buf],
            )

    def wait_wb(buf):
        for f_blk in range(4):
            pltpu.make_async_copy(
                t_v.at[buf, pl.ds(f_blk * 32, 32), pl.ds(0, 128)],
                out_hbm.at[0, pl.ds(f_blk * 1024, 32), :],
                wsem.at[buf],
            ).wait()

    pltpu.async_copy(
        idx_hbm.at[pl.ds(col0, COLS), :], idx_blk, isem
    ).wait()

    extract_idx(0, 0)
    issue_gather(0)

    def body(g, carry):
        for buf in range(2):
            i = g * 2 + buf
            wait_gather(buf)

            @pl.when(i + 1 < B1)
            def _():
                extract_idx(i + 1, 1 - buf)
                issue_gather(1 - buf)

            @pl.when(i >= 2)
            def _():
                wait_wb(buf)

            transpose(buf)
            issue_wb(i, buf)
        return carry

    lax.fori_loop(0, B1 // 2, body, 0)
    for buf in range(2):
        wait_wb(buf)


def kernel(input, weight):
    idx = input.astype(jnp.int32)

    mesh = plsc.VectorSubcoreMesh(core_axis_name="c", subcore_axis_name="s")
    run = pl.kernel(
        _embed_kernel,
        mesh=mesh,
        out_type=jax.ShapeDtypeStruct((B1, 4096, 128), jnp.float32),
        scratch_types=[
            pltpu.VMEM((COLS, B1), jnp.int32),
            pltpu.VMEM((2, COLS), jnp.int32),
            pltpu.VMEM((2, COLS, EMBED_DIM), jnp.float32),
            pltpu.VMEM((2, 128, TPAD), jnp.float32),
            pltpu.SemaphoreType.DMA,
            pltpu.SemaphoreType.DMA((2,)),
            pltpu.SemaphoreType.DMA((2,)),
        ],
        compiler_params=pltpu.CompilerParams(
            use_tc_tiling_on_sc=False, needs_layout_passes=False
        ),
    )
    out3d = run(idx, weight)
    return (
        out3d.reshape(B1, 4, 128, 8, 128)
        .transpose(2, 4, 0, 1, 3)
        .reshape(B0, B1, EMBED_DIM)
    )
